# Initial kernel scaffold; baseline (speedup 1.0000x reference)
import jax
import jax.numpy as jnp
from jax.experimental import pallas as pl


def kernel(x, conv_w, conv_b, fc_w, fc_b):
    raise NotImplementedError("write your pallas kernel here")


# speedup vs baseline: 6.6466x; 6.6466x over previous
import functools

import numpy as np
import jax
import jax.numpy as jnp
from jax.experimental import pallas as pl
from jax.experimental.pallas import tpu as pltpu

C_IN, C_OUT, KS, POOL = 3, 8, 3, 4
H = W = 48
HP = WP = H // POOL
N_CLASSES = 43
LANES_IN = C_IN * W
LANES_CONV = C_OUT * W
POOLED = C_OUT * WP
CAND_STRIDE = 128

_shift_np = np.zeros((KS, W, W), np.float32)
for _kx in range(KS):
    for _wo in range(W):
        _wi = _wo + _kx - 1
        if 0 <= _wi < W:
            _shift_np[_kx, _wi, _wo] = 1.0

_selc_np = np.zeros((LANES_CONV, 4 * CAND_STRIDE), np.float32)
for _j in range(POOL):
    for _co in range(C_OUT):
        for _wp in range(WP):
            _selc_np[_co * W + POOL * _wp + _j,
                     CAND_STRIDE * _j + _co * WP + _wp] = 1.0


def _cnn_kernel(xr_ref, wky_ref, selc_ref, b96_ref, wf_ref, bf_ref, out_ref):
    bblk = out_ref.shape[0]
    rows = bblk * H
    xt = xr_ref[...].reshape(rows, LANES_IN)

    hmod = jax.lax.broadcasted_iota(jnp.int32, (rows, 1), 0) % H
    wky = wky_ref[...]
    acc = jnp.dot(xt, wky[1], preferred_element_type=jnp.float32)
    z0 = jnp.dot(xt, wky[0], preferred_element_type=jnp.float32)
    sh0 = jnp.concatenate([jnp.zeros((1, LANES_CONV), jnp.float32), z0[:-1]],
                          axis=0)
    acc = acc + jnp.where(hmod == 0, 0.0, sh0)
    z2 = jnp.dot(xt, wky[2], preferred_element_type=jnp.float32)
    sh2 = jnp.concatenate([z2[1:], jnp.zeros((1, LANES_CONV), jnp.float32)],
                          axis=0)
    acc = acc + jnp.where(hmod == H - 1, 0.0, sh2)

    r12 = acc.reshape(rows // POOL, POOL, LANES_CONV).max(axis=1)

    cand = jnp.dot(r12, selc_ref[...], preferred_element_type=jnp.float32)
    pooled = jnp.maximum(
        jnp.maximum(cand[:, 0:POOLED],
                    cand[:, CAND_STRIDE:CAND_STRIDE + POOLED]),
        jnp.maximum(cand[:, 2 * CAND_STRIDE:2 * CAND_STRIDE + POOLED],
                    cand[:, 3 * CAND_STRIDE:3 * CAND_STRIDE + POOLED]))
    act = jnp.maximum(pooled + b96_ref[...], 0.0)

    logits = jnp.broadcast_to(bf_ref[...], (bblk, N_CLASSES))
    wf = wf_ref[...]
    act3 = act.reshape(bblk, HP, POOLED)
    for hp in range(HP):
        logits = logits + jnp.dot(act3[:, hp, :], wf[hp],
                                  preferred_element_type=jnp.float32)
    out_ref[...] = logits


@functools.partial(jax.jit, static_argnames=("block_b",))
def _forward(x, conv_w, conv_b, fc_w, fc_b, block_b=128):
    B = x.shape[0]
    bblk = block_b
    while B % bblk:
        bblk //= 2
    n_steps = B // bblk

    xr = x.transpose(0, 2, 1, 3).reshape(B, H, LANES_IN).astype(jnp.bfloat16)

    wky = jnp.einsum("xab,ocyx->ycaob", jnp.asarray(_shift_np),
                     conv_w.astype(jnp.float32))
    wky = wky.reshape(KS, LANES_IN, LANES_CONV).astype(jnp.bfloat16)

    b96 = jnp.repeat(conv_b.astype(jnp.float32), WP).reshape(1, POOLED)
    wf = (fc_w.astype(jnp.float32).reshape(N_CLASSES, C_OUT, HP, WP)
          .transpose(2, 1, 3, 0).reshape(HP, POOLED, N_CLASSES))
    bf2 = fc_b.astype(jnp.float32).reshape(1, N_CLASSES)

    return pl.pallas_call(
        _cnn_kernel,
        out_shape=jax.ShapeDtypeStruct((B, N_CLASSES), jnp.float32),
        grid=(n_steps,),
        in_specs=[
            pl.BlockSpec((bblk, H, LANES_IN), lambda s: (s, 0, 0)),
            pl.BlockSpec((KS, LANES_IN, LANES_CONV), lambda s: (0, 0, 0)),
            pl.BlockSpec((LANES_CONV, 4 * CAND_STRIDE), lambda s: (0, 0)),
            pl.BlockSpec((1, POOLED), lambda s: (0, 0)),
            pl.BlockSpec((HP, POOLED, N_CLASSES), lambda s: (0, 0, 0)),
            pl.BlockSpec((1, N_CLASSES), lambda s: (0, 0)),
        ],
        out_specs=pl.BlockSpec((bblk, N_CLASSES), lambda s: (s, 0)),
        compiler_params=pltpu.CompilerParams(
            dimension_semantics=("parallel",)),
    )(xr, wky, jnp.asarray(_selc_np), b96, wf, bf2)


def kernel(x, conv_w, conv_b, fc_w, fc_b):
    return _forward(x, conv_w, conv_b, fc_w, fc_b)


# speedup vs baseline: 8.0145x; 1.2058x over previous
import functools

import numpy as np
import jax
import jax.numpy as jnp
from jax.experimental import pallas as pl
from jax.experimental.pallas import tpu as pltpu

C_IN, C_OUT, KS, POOL = 3, 8, 3, 4
H = W = 48
HP = WP = H // POOL
N_CLASSES = 43
LANES_IN = C_IN * W
LANES_CONV = C_OUT * W
POOLED = C_OUT * WP
CAND_STRIDE = 128

_shift_np = np.zeros((KS, W, W), np.float32)
for _kx in range(KS):
    for _wo in range(W):
        _wi = _wo + _kx - 1
        if 0 <= _wi < W:
            _shift_np[_kx, _wi, _wo] = 1.0

_selc_np = np.zeros((LANES_CONV, 4 * CAND_STRIDE), np.float32)
for _j in range(POOL):
    for _co in range(C_OUT):
        for _wp in range(WP):
            _selc_np[_co * W + POOL * _wp + _j,
                     CAND_STRIDE * _j + _co * WP + _wp] = 1.0


def _cnn_kernel(xr_ref, wky_ref, selc_ref, b96_ref, wf_ref, bf_ref, out_ref):
    bblk = out_ref.shape[0]
    nb = HP * bblk
    xt = xr_ref[...].reshape(POOL * nb, LANES_IN)

    wky = wky_ref[...]
    z0 = jnp.dot(xt, wky[0], preferred_element_type=jnp.float32)
    z1 = jnp.dot(xt, wky[1], preferred_element_type=jnp.float32)
    z2 = jnp.dot(xt, wky[2], preferred_element_type=jnp.float32)
    zb0 = [z0[k * nb:(k + 1) * nb] for k in range(POOL)]
    zb1 = [z1[k * nb:(k + 1) * nb] for k in range(POOL)]
    zb2 = [z2[k * nb:(k + 1) * nb] for k in range(POOL)]

    hpmod = jax.lax.broadcasted_iota(jnp.int32, (nb, 1), 0) % HP
    zrow = jnp.zeros((1, LANES_CONV), jnp.float32)
    sd = jnp.concatenate([zrow, zb0[3][:-1]], axis=0)
    sd = jnp.where(hpmod == 0, 0.0, sd)
    su = jnp.concatenate([zb2[0][1:], zrow], axis=0)
    su = jnp.where(hpmod == HP - 1, 0.0, su)

    conv0 = sd + zb1[0] + zb2[1]
    conv1 = zb0[0] + zb1[1] + zb2[2]
    conv2 = zb0[1] + zb1[2] + zb2[3]
    conv3 = zb0[2] + zb1[3] + su

    r12 = jnp.maximum(jnp.maximum(conv0, conv1),
                      jnp.maximum(conv2, conv3))

    cand = jnp.dot(r12, selc_ref[...], preferred_element_type=jnp.float32)
    pooled = jnp.maximum(
        jnp.maximum(cand[:, 0:POOLED],
                    cand[:, CAND_STRIDE:CAND_STRIDE + POOLED]),
        jnp.maximum(cand[:, 2 * CAND_STRIDE:2 * CAND_STRIDE + POOLED],
                    cand[:, 3 * CAND_STRIDE:3 * CAND_STRIDE + POOLED]))
    act = jnp.maximum(pooled + b96_ref[...], 0.0)

    logits = jnp.broadcast_to(bf_ref[...], (bblk, N_CLASSES))
    wf = wf_ref[...]
    act3 = act.reshape(bblk, HP, POOLED)
    for hp in range(HP):
        logits = logits + jnp.dot(act3[:, hp, :], wf[hp],
                                  preferred_element_type=jnp.float32)
    out_ref[...] = logits


@functools.partial(jax.jit, static_argnames=("block_b",))
def _forward(x, conv_w, conv_b, fc_w, fc_b, block_b=128):
    B = x.shape[0]
    bblk = block_b
    while B % bblk:
        bblk //= 2
    n_steps = B // bblk

    xr = (x.transpose(0, 2, 1, 3).reshape(B, HP, POOL, LANES_IN)
          .transpose(2, 0, 1, 3).reshape(POOL, B * HP, LANES_IN)
          .astype(jnp.bfloat16))

    wky = jnp.einsum("xab,ocyx->ycaob", jnp.asarray(_shift_np),
                     conv_w.astype(jnp.float32))
    wky = wky.reshape(KS, LANES_IN, LANES_CONV).astype(jnp.bfloat16)

    b96 = jnp.repeat(conv_b.astype(jnp.float32), WP).reshape(1, POOLED)
    wf = (fc_w.astype(jnp.float32).reshape(N_CLASSES, C_OUT, HP, WP)
          .transpose(2, 1, 3, 0).reshape(HP, POOLED, N_CLASSES))
    bf2 = fc_b.astype(jnp.float32).reshape(1, N_CLASSES)

    return pl.pallas_call(
        _cnn_kernel,
        out_shape=jax.ShapeDtypeStruct((B, N_CLASSES), jnp.float32),
        grid=(n_steps,),
        in_specs=[
            pl.BlockSpec((POOL, HP * bblk, LANES_IN), lambda s: (0, s, 0)),
            pl.BlockSpec((KS, LANES_IN, LANES_CONV), lambda s: (0, 0, 0)),
            pl.BlockSpec((LANES_CONV, 4 * CAND_STRIDE), lambda s: (0, 0)),
            pl.BlockSpec((1, POOLED), lambda s: (0, 0)),
            pl.BlockSpec((HP, POOLED, N_CLASSES), lambda s: (0, 0, 0)),
            pl.BlockSpec((1, N_CLASSES), lambda s: (0, 0)),
        ],
        out_specs=pl.BlockSpec((bblk, N_CLASSES), lambda s: (s, 0)),
        compiler_params=pltpu.CompilerParams(
            dimension_semantics=("parallel",)),
    )(xr, wky, jnp.asarray(_selc_np), b96, wf, bf2)


def kernel(x, conv_w, conv_b, fc_w, fc_b):
    return _forward(x, conv_w, conv_b, fc_w, fc_b)
